# baseline (speedup 1.0000x reference)
import functools

import jax
import jax.numpy as jnp
from jax import lax
from jax.experimental import pallas as pl
from jax.experimental.pallas import tpu as pltpu
from jax.experimental.pallas import tpu_sc as plsc

_B = 16384
_D = 64
_DP = 128
_N = 100000
_NP = _N // 2
_NW = 32
_BPW = _B // _NW
_IC = 128
_NC = _BPW // _IC
_HALF = _BPW // 2
_NBLK = 782
_RNDS = (_NBLK + _NW - 1) // _NW

_mesh = plsc.VectorSubcoreMesh(core_axis_name="c", subcore_axis_name="s")
_params = pltpu.CompilerParams(needs_layout_passes=False)


def _build_transpose():
    @functools.partial(
        pl.kernel,
        mesh=_mesh,
        compiler_params=_params,
        out_type=(jax.ShapeDtypeStruct((_NP, _DP), jnp.float32),
                  jax.ShapeDtypeStruct((_NP, _DP), jnp.float32)),
        scratch_types=[
            pltpu.VMEM((_D, _DP), jnp.float32),
            pltpu.VMEM((_D, _DP), jnp.float32),
            pltpu.VMEM((_D, _DP), jnp.float32),
            pltpu.VMEM((_D, _DP), jnp.float32),
            pltpu.SemaphoreType.DMA,
            pltpu.SemaphoreType.DMA,
            pltpu.SemaphoreType.DMA,
            pltpu.SemaphoreType.DMA,
        ],
    )
    def body(uet_hbm, iet_hbm, utail_hbm, itail_hbm, up_hbm, ip_hbm,
             cbuf0, cbuf1, obuf0, obuf1, sem0, sem1, osem0, osem1):
        wid = lax.axis_index("s") * 2 + lax.axis_index("c")
        lanes = lax.iota(jnp.int32, 16)
        cbufs = (cbuf0, cbuf1)
        obufs = (obuf0, obuf1)
        sems = (sem0, sem1)
        osems = (osem0, osem1)
        srcs = (uet_hbm, iet_hbm)
        dsts = (up_hbm, ip_hbm)

        _NU = 2 * (_NBLK - 1)

        def fire(u, p):
            blk = lax.shift_right_logical(u, 1)
            i0 = pl.multiple_of(blk * _DP, _DP)
            tab = lax.bitwise_and(u, 1)
            for t in range(2):
                @pl.when(tab == t)
                def _():
                    for t8 in range(_D // 8):
                        pltpu.async_copy(
                            srcs[t].at[pl.ds(t8 * 8, 8), pl.ds(i0, _DP)],
                            cbufs[p].at[pl.ds(t8 * 8, 8), :], sems[p])

        dummy = uet_hbm.at[pl.ds(0, _D), pl.ds(0, _DP)]

        def wait_in(p):
            pltpu.make_async_copy(dummy, cbufs[p], sems[p]).wait()

        def transpose(p):
            def sub(t, carry):
                ib = lax.shift_right_logical(t, 2)
                db = lax.bitwise_and(t, 3)
                for j in range(16):
                    dd = lax.bitwise_and(lanes + j, 15)
                    rows = db * 16 + dd
                    cols = ib * 16 + lanes
                    v = plsc.load_gather(cbufs[p], [rows, cols])
                    flat = (ib * 16 + lanes) * _D + db * 16 + dd
                    plsc.store_scatter(
                        obufs[p],
                        [lax.shift_right_logical(flat, 7),
                         lax.bitwise_and(flat, _DP - 1)], v)
                return carry

            lax.fori_loop(0, (_DP // 16) * (_D // 16), sub, 0, unroll=4)

        def fire_out(u, p):
            blk = lax.shift_right_logical(u, 1)
            r0 = pl.multiple_of(blk * (_DP // 2), _D)
            tab = lax.bitwise_and(u, 1)
            for t in range(2):
                @pl.when(tab == t)
                def _():
                    pltpu.async_copy(obufs[p], dsts[t].at[pl.ds(r0, _D), :],
                                     osems[p])

        def wait_out(p):
            pltpu.make_async_copy(dummy, obufs[p], osems[p]).wait()

        @pl.when(wid < _NU)
        def _():
            fire(wid, 0)

        @pl.when(wid + _NW < _NU)
        def _():
            fire(wid + _NW, 1)

        def round_(t, carry):
            for p in range(2):
                u = (2 * t + p) * _NW + wid

                @pl.when(u < _NU)
                def _():
                    wait_in(p)

                    @pl.when(u >= 2 * _NW)
                    def _():
                        wait_out(p)

                    transpose(p)
                    fire_out(u, p)
                    nxt = u + 2 * _NW

                    @pl.when(nxt < _NU)
                    def _():
                        fire(nxt, p)

            return carry

        lax.fori_loop(0, (_NU + 2 * _NW - 1) // (2 * _NW), round_, 0)

        @pl.when(wid < _NU)
        def _():
            wait_out(0)

        @pl.when(wid + _NW < _NU)
        def _():
            wait_out(1)

        @pl.when(wid == 13)
        def _():
            tsl = pl.ds(0, 16)
            dsl = pl.ds(_NP - 16, 16)
            pltpu.sync_copy(utail_hbm, cbuf0.at[tsl, :])
            pltpu.sync_copy(cbuf0.at[tsl, :], up_hbm.at[dsl, :])
            pltpu.sync_copy(itail_hbm, cbuf0.at[tsl, :])
            pltpu.sync_copy(cbuf0.at[tsl, :], ip_hbm.at[dsl, :])

    return body


def _build_gather():
    @functools.partial(
        pl.kernel,
        mesh=_mesh,
        compiler_params=_params,
        out_type=jax.ShapeDtypeStruct((_B,), jnp.float32),
        scratch_types=[
            pltpu.VMEM((_BPW,), jnp.int32),
            pltpu.VMEM((_BPW,), jnp.int32),
            pltpu.VMEM((_BPW,), jnp.int32),
            pltpu.VMEM((_BPW,), jnp.int32),
            pltpu.VMEM((_HALF, _DP), jnp.float32),
            pltpu.VMEM((_HALF, _DP), jnp.float32),
            pltpu.VMEM((_BPW,), jnp.float32),
            pltpu.VMEM((_BPW,), jnp.float32),
            pltpu.VMEM((_BPW,), jnp.float32),
            pltpu.SemaphoreType.DMA,
            pltpu.SemaphoreType.DMA,
        ],
    )
    def body(user_hbm, item_hbm, uemb_hbm, iemb_hbm, ubias_hbm, ibias_hbm,
             out_hbm, uidx, iidx, ublk, iblk, urows, irows, ub, ib, outv,
             sem, bsem):
        wid = lax.axis_index("s") * 2 + lax.axis_index("c")
        base = wid * _BPW

        pltpu.sync_copy(user_hbm.at[pl.ds(base, _BPW)], uidx)
        pltpu.sync_copy(item_hbm.at[pl.ds(base, _BPW)], iidx)

        for k in range(_BPW // 16):
            s16 = pl.ds(k * 16, 16)
            ublk[s16] = lax.shift_right_logical(uidx[s16], 1)
            iblk[s16] = lax.shift_right_logical(iidx[s16], 1)

        bias_copies = []
        for j in range(_NC):
            sl = pl.ds(j * _IC, _IC)
            bias_copies.append(
                pltpu.async_copy(ubias_hbm.at[uidx.at[sl]], ub.at[sl], bsem))
            bias_copies.append(
                pltpu.async_copy(ibias_hbm.at[iidx.at[sl]], ib.at[sl], bsem))

        lanes = lax.iota(jnp.int32, 16)

        def fire(h):
            cs = []
            for j in range(_HALF // _IC):
                isl = pl.ds(h * _HALF + j * _IC, _IC)
                dsl = pl.ds(j * _IC, _IC)
                cs.append(pltpu.async_copy(uemb_hbm.at[ublk.at[isl]],
                                           urows.at[dsl], sem))
                cs.append(pltpu.async_copy(iemb_hbm.at[iblk.at[isl]],
                                           irows.at[dsl], sem))
            return cs

        def compute(h):
            def group(g, carry):
                rows = lanes + g * 16
                sl16 = pl.ds(h * _HALF + g * 16, 16)
                ucol = lax.shift_left(lax.bitwise_and(uidx[sl16], 1), 6)
                icol = lax.shift_left(lax.bitwise_and(iidx[sl16], 1), 6)
                acc = ub[sl16] + ib[sl16] + 3.5
                for d in range(_D):
                    rot = lax.bitwise_and(lanes + d, _D - 1)
                    acc = acc + (
                        plsc.load_gather(urows, [rows, ucol + rot])
                        * plsc.load_gather(irows, [rows, icol + rot]))
                outv[sl16] = acc
                return carry

            lax.fori_loop(0, _HALF // 16, group, 0)

        for c in bias_copies:
            c.wait()

        for h in range(2):
            for c in fire(h):
                c.wait()
            compute(h)

        pltpu.sync_copy(outv, out_hbm.at[pl.ds(base, _BPW)])

    return body


_sc_transpose = _build_transpose()
_sc_gather = _build_gather()


def kernel(user, item, user_emb, item_emb, user_bias, item_bias):
    utail = user_emb[_N - 32:].reshape(16, _DP)
    itail = item_emb[_N - 32:].reshape(16, _DP)
    up, ip = _sc_transpose(user_emb.T, item_emb.T, utail, itail)
    return _sc_gather(user.astype(jnp.int32), item.astype(jnp.int32),
                      up, ip, user_bias.sum(axis=1), item_bias.sum(axis=1))


# speedup vs baseline: 1.2748x; 1.2748x over previous
import functools

import jax
import jax.numpy as jnp
from jax import lax
from jax.experimental import pallas as pl
from jax.experimental.pallas import tpu as pltpu
from jax.experimental.pallas import tpu_sc as plsc

_B = 16384
_D = 64
_DP = 128
_NW = 32
_BPW = _B // _NW
_IC = 128
_NC = _BPW // _IC
_HALF = _BPW // 2


def _build():
    mesh = plsc.VectorSubcoreMesh(core_axis_name="c", subcore_axis_name="s")

    @functools.partial(
        pl.kernel,
        mesh=mesh,
        compiler_params=pltpu.CompilerParams(needs_layout_passes=False),
        out_type=jax.ShapeDtypeStruct((_B,), jnp.float32),
        scratch_types=[
            pltpu.VMEM((_BPW,), jnp.int32),
            pltpu.VMEM((_BPW,), jnp.int32),
            pltpu.VMEM((_HALF, _DP), jnp.float32),
            pltpu.VMEM((_HALF, _DP), jnp.float32),
            pltpu.VMEM((_BPW,), jnp.float32),
            pltpu.VMEM((_BPW,), jnp.float32),
            pltpu.VMEM((_BPW,), jnp.float32),
            pltpu.SemaphoreType.DMA,
            pltpu.SemaphoreType.DMA,
        ],
    )
    def body(user_hbm, item_hbm, uemb_hbm, iemb_hbm, ubias_hbm, ibias_hbm,
             out_hbm, uidx, iidx, urows, irows, ub, ib, outv, sem, bsem):
        wid = lax.axis_index("s") * 2 + lax.axis_index("c")
        base = wid * _BPW

        pltpu.sync_copy(user_hbm.at[pl.ds(base, _BPW)], uidx)
        pltpu.sync_copy(item_hbm.at[pl.ds(base, _BPW)], iidx)

        bias_copies = []
        for j in range(_NC):
            sl = pl.ds(j * _IC, _IC)
            bias_copies.append(
                pltpu.async_copy(ubias_hbm.at[uidx.at[sl]], ub.at[sl], bsem))
            bias_copies.append(
                pltpu.async_copy(ibias_hbm.at[iidx.at[sl]], ib.at[sl], bsem))

        lanes = lax.iota(jnp.int32, 16)

        def fire(h):
            cs = []
            for j in range(_HALF // _IC):
                isl = pl.ds(h * _HALF + j * _IC, _IC)
                dsl = pl.ds(j * _IC, _IC)
                cs.append(pltpu.async_copy(uemb_hbm.at[uidx.at[isl]],
                                           urows.at[dsl], sem))
                cs.append(pltpu.async_copy(iemb_hbm.at[iidx.at[isl]],
                                           irows.at[dsl], sem))
            return cs

        def compute(h):
            def group(g, carry):
                rows = lanes + g * 16
                sl16 = pl.ds(h * _HALF + g * 16, 16)
                acc = ub[sl16] + ib[sl16] + 3.5
                for d in range(_D):
                    cols = lax.bitwise_and(lanes + d, _D - 1)
                    acc = acc + (plsc.load_gather(urows, [rows, cols])
                                 * plsc.load_gather(irows, [rows, cols]))
                outv[sl16] = acc
                return carry

            lax.fori_loop(0, _HALF // 16, group, 0)

        for c in bias_copies:
            c.wait()

        for h in range(2):
            for c in fire(h):
                c.wait()
            compute(h)

        pltpu.sync_copy(outv, out_hbm.at[pl.ds(base, _BPW)])

    return body


_sc_call = _build()


def kernel(user, item, user_emb, item_emb, user_bias, item_bias):
    up = jnp.pad(user_emb, ((0, 0), (0, _DP - _D)))
    ip = jnp.pad(item_emb, ((0, 0), (0, _DP - _D)))
    return _sc_call(user.astype(jnp.int32), item.astype(jnp.int32),
                    up, ip, user_bias.sum(axis=1), item_bias.sum(axis=1))
